# baseline (speedup 1.0000x reference)
import functools

import jax
import jax.numpy as jnp
from jax import lax
from jax.experimental import pallas as pl
from jax.experimental.pallas import tpu as pltpu
from jax.experimental.pallas import tpu_sc as plsc

B = 4096
T = 77
D = 128
NC = 2
NS = 16
NW = NC * NS
SEQ_PER_W = B // NW
NBUF = 4


def _body(tok_hbm, table_hbm, out_hbm,
          idx_v, bufs,
          sg0, sg1, sg2, sg3, ss0, ss1, ss2, ss3):
  sem_g = (sg0, sg1, sg2, sg3)
  sem_s = (ss0, ss1, ss2, ss3)
  wid = lax.axis_index("s") * NC + lax.axis_index("c")
  seq0 = wid * SEQ_PER_W

  pltpu.sync_copy(tok_hbm.at[pl.ds(seq0, SEQ_PER_W)], idx_v)

  def gather(s, b):
    return pltpu.make_async_copy(table_hbm.at[idx_v.at[s]], bufs.at[b],
                                 sem_g[b])

  def scatter(s, b):
    return pltpu.make_async_copy(bufs.at[b], out_hbm.at[seq0 + s], sem_s[b])

  def step(s, b, refill, drain):
    b2 = (b + 2) % NBUF
    if refill:
      if drain:
        scatter(s - 2, b2).wait()
      gather(s + 2, b2).start()
    gather(s, b).wait()
    scatter(s, b).start()

  gather(0, 0).start()
  gather(1, 1).start()

  for b in range(NBUF):
    step(b, b, refill=True, drain=(b >= 2))

  def outer(g, carry):
    for b in range(NBUF):
      step(g * NBUF + b, b, refill=True, drain=True)
    return carry

  lax.fori_loop(1, SEQ_PER_W // NBUF - 1, outer, 0)

  g = SEQ_PER_W // NBUF - 1
  for b in range(NBUF):
    step(g * NBUF + b, b, refill=(b < 2), drain=(b < 2))

  for b in range(NBUF):
    scatter(g * NBUF + b, b).wait()


_kern = functools.partial(
    pl.kernel,
    out_type=jax.ShapeDtypeStruct((B, T, D), jnp.float32),
    mesh=plsc.VectorSubcoreMesh(core_axis_name="c", subcore_axis_name="s"),
    scratch_types=[
        pltpu.VMEM((SEQ_PER_W, T), jnp.int32),
        pltpu.VMEM((NBUF, T, D), jnp.float32),
    ] + [pltpu.SemaphoreType.DMA] * (2 * NBUF),
)(_body)


from jax.experimental import layout as _jl


def _impl(tokens, token_embedding, position_embedding):
  gathered = _kern(tokens, token_embedding)
  out = gathered + position_embedding[None, :, :]
  lay = _jl.Layout(major_to_minor=(0, 1, 2), tiling=((8, 128),))
  return _jl.with_layout_constraint(out, lay)


def kernel(tokens, token_embedding, position_embedding):
  return jax.jit(_impl)(tokens, token_embedding, position_embedding)


# speedup vs baseline: 1.4756x; 1.4756x over previous
import functools

import jax
import jax.numpy as jnp
from jax import lax
from jax.experimental import pallas as pl
from jax.experimental.pallas import tpu as pltpu
from jax.experimental.pallas import tpu_sc as plsc

B = 4096
T = 77
D = 128
NC = 2
NS = 16
NW = NC * NS
SEQ_PER_W = B // NW
NBUF = 4
TPAD = 80


def _gather_body(tok_hbm, table_hbm, out_hbm,
                 idx_v, bufs,
                 sg0, sg1, sg2, sg3, ss0, ss1, ss2, ss3):
  sem_g = (sg0, sg1, sg2, sg3)
  sem_s = (ss0, ss1, ss2, ss3)
  wid = lax.axis_index("s") * NC + lax.axis_index("c")
  seq0 = wid * SEQ_PER_W

  pltpu.sync_copy(tok_hbm.at[pl.ds(seq0, SEQ_PER_W)], idx_v)

  def gather(s, b):
    return pltpu.make_async_copy(table_hbm.at[idx_v.at[s]],
                                 bufs.at[b, pl.ds(0, T)], sem_g[b])

  def scatter(s, b):
    return pltpu.make_async_copy(
        bufs.at[b], out_hbm.at[pl.ds((seq0 + s) * TPAD, TPAD)], sem_s[b])

  def step(s, b, refill, drain):
    b2 = (b + 2) % NBUF
    if refill:
      if drain:
        scatter(s - 2, b2).wait()
      gather(s + 2, b2).start()
    gather(s, b).wait()
    scatter(s, b).start()

  gather(0, 0).start()
  gather(1, 1).start()

  for b in range(NBUF):
    step(b, b, refill=True, drain=(b >= 2))

  def outer(g, carry):
    for b in range(NBUF):
      step(g * NBUF + b, b, refill=True, drain=True)
    return carry

  lax.fori_loop(1, SEQ_PER_W // NBUF - 1, outer, 0)

  g = SEQ_PER_W // NBUF - 1
  for b in range(NBUF):
    step(g * NBUF + b, b, refill=(b < 2), drain=(b < 2))

  for b in range(NBUF):
    scatter(g * NBUF + b, b).wait()


_gather = functools.partial(
    pl.kernel,
    out_type=jax.ShapeDtypeStruct((B * TPAD, D), jnp.float32),
    mesh=plsc.VectorSubcoreMesh(core_axis_name="c", subcore_axis_name="s"),
    scratch_types=[
        pltpu.VMEM((SEQ_PER_W, T), jnp.int32),
        pltpu.VMEM((NBUF, TPAD, D), jnp.float32),
    ] + [pltpu.SemaphoreType.DMA] * (2 * NBUF),
)(_gather_body)


@jax.jit
def kernel(tokens, token_embedding, position_embedding):
  gathered = _gather(tokens, token_embedding)
  g3 = gathered.reshape(B, TPAD, D)
  return g3[:, :T, :] + position_embedding[None, :, :]
